# baseline (speedup 1.0000x reference)
import jax
import jax.numpy as jnp
from jax.experimental import pallas as pl
from jax.experimental.pallas import tpu as pltpu

N = 10000
E = 160000
IN = 256
H = 4
C = 256
HC = H * C

_ROW_BLK = 1000


def _mm_body(x_ref, w_ref, asrc_ref, adst_ref, xw_ref, a_ref):
    xw = jnp.dot(x_ref[...], w_ref[...], preferred_element_type=jnp.float32)
    xw_ref[...] = xw
    parts = []
    for att_ref in (asrc_ref, adst_ref):
        t = xw * att_ref[...]
        for h in range(H):
            parts.append(jnp.sum(t[:, h * C:(h + 1) * C], axis=1, keepdims=True))
    a_ref[...] = jnp.concatenate(parts, axis=1)


def _dense_stage(x, W, att_src_flat, att_dst_flat):
    grid = N // _ROW_BLK
    xw, a = pl.pallas_call(
        _mm_body,
        grid=(grid,),
        in_specs=[
            pl.BlockSpec((_ROW_BLK, IN), lambda i: (i, 0)),
            pl.BlockSpec((IN, HC), lambda i: (0, 0)),
            pl.BlockSpec((1, HC), lambda i: (0, 0)),
            pl.BlockSpec((1, HC), lambda i: (0, 0)),
        ],
        out_specs=[
            pl.BlockSpec((_ROW_BLK, HC), lambda i: (i, 0)),
            pl.BlockSpec((_ROW_BLK, 2 * H), lambda i: (i, 0)),
        ],
        out_shape=[
            jax.ShapeDtypeStruct((N, HC), jnp.float32),
            jax.ShapeDtypeStruct((N, 2 * H), jnp.float32),
        ],
    )(x, W, att_src_flat, att_dst_flat)
    return xw, a[:, :H], a[:, H:]


def kernel(x, edge_index, W, att_src, att_dst, bias, prelu_a):
    xw, a_src, a_dst = _dense_stage(
        x, W, att_src.reshape(1, HC), att_dst.reshape(1, HC))

    loop = jnp.arange(N, dtype=edge_index.dtype)
    src = jnp.concatenate([edge_index[0], loop])
    dst = jnp.concatenate([edge_index[1], loop])
    alpha = a_src[src] + a_dst[dst]
    alpha = jax.nn.leaky_relu(alpha, 0.2)
    amax = jax.ops.segment_max(alpha, dst, num_segments=N)
    amax = jnp.where(jnp.isfinite(amax), amax, 0.0)
    ex = jnp.exp(alpha - amax[dst])
    denom = jax.ops.segment_sum(ex, dst, num_segments=N)
    att = ex / (denom[dst] + 1e-16)
    msg = xw.reshape(N, H, C)[src] * att[:, :, None]
    out = jax.ops.segment_sum(msg, dst, num_segments=N)
    out = out.reshape(N, HC) + bias
    return jnp.where(out >= 0, out, prelu_a * out)


# speedup vs baseline: 4.6949x; 4.6949x over previous
import dataclasses
import functools

import jax
import jax.numpy as jnp
from jax import lax
from jax.experimental import pallas as pl
from jax.experimental.pallas import tpu as pltpu
from jax.experimental.pallas import tpu_sc as plsc

N = 10000
E = 160000
IN = 256
H = 4
C = 256
HC = H * C

NC = 2
NS = 16
NT = NC * NS
L = 16

OWN = 80
NPASS = 4
ROWS_PAD = NPASS * NT * OWN
ACCW = HC + 128
DCOL = HC
AUXW = 128
EBLK = 2000
NBLK = E // EBLK
SELCAP = EBLK + L

_ROW_BLK = 1000


def _mm_body(x_ref, w_ref, asrc_ref, adst_ref, xw_ref, aux_ref):
    xw = jnp.dot(x_ref[...], w_ref[...], preferred_element_type=jnp.float32)
    xw_ref[...] = xw
    cols = []
    for att_ref in (asrc_ref, adst_ref):
        t = xw * att_ref[...]
        for h in range(H):
            cols.append(jnp.sum(t[:, h * C:(h + 1) * C], axis=1, keepdims=True))
    a_s = jnp.concatenate(cols[:H], axis=1)
    a_d = jnp.concatenate(cols[H:], axis=1)
    t = a_s + a_d
    s = jnp.maximum(t, 0.0) + 0.2 * jnp.minimum(t, 0.0)
    pad = jnp.zeros((a_s.shape[0], AUXW - 3 * H), jnp.float32)
    aux_ref[...] = jnp.concatenate([a_s, a_d, s, pad], axis=1)


def _dense_stage(x, W, att_src_flat, att_dst_flat):
    return pl.pallas_call(
        _mm_body,
        grid=(N // _ROW_BLK,),
        in_specs=[
            pl.BlockSpec((_ROW_BLK, IN), lambda i: (i, 0)),
            pl.BlockSpec((IN, HC), lambda i: (0, 0)),
            pl.BlockSpec((1, HC), lambda i: (0, 0)),
            pl.BlockSpec((1, HC), lambda i: (0, 0)),
        ],
        out_specs=[
            pl.BlockSpec((_ROW_BLK, HC), lambda i: (i, 0)),
            pl.BlockSpec((_ROW_BLK, AUXW), lambda i: (i, 0)),
        ],
        out_shape=[
            jax.ShapeDtypeStruct((ROWS_PAD, HC), jnp.float32),
            jax.ShapeDtypeStruct((N, AUXW), jnp.float32),
        ],
    )(x, W, att_src_flat, att_dst_flat)


def _sc_edge_body(xw_hbm, aux_hbm, src_hbm, dst_hbm, accx_out, accd_out,
                  ebuf_src, ebuf_dst, sel_src, sel_dst, rowbuf, idxbuf,
                  aux1, aux2, wbuf, sem_row, sem_aux, accx, accd):
    c = lax.axis_index("c")
    s = lax.axis_index("s")
    tid = s * NC + c
    lane = lax.iota(jnp.int32, L)

    zero16i = jnp.zeros((L,), jnp.int32)
    zero16f = jnp.zeros((L,), jnp.float32)
    dpat = jnp.where(lane < H, 1.0, 0.0).astype(jnp.float32)

    @pl.loop(0, SELCAP // L)
    def _z(i):
        plsc.store_scatter(sel_src, [i * L + lane], zero16i)
        plsc.store_scatter(sel_dst, [i * L + lane], zero16i)

    @pl.loop(0, NPASS)
    def _pass(p):
        own_base = (p * NT + tid) * OWN

        pltpu.sync_copy(xw_hbm.at[pl.ds(own_base * 8, OWN * 8)], accx)

        @pl.loop(0, OWN)
        def _dinit(r):
            for q in range(128 // L):
                accd[r, pl.ds(q * L, L)] = dpat if q == 0 else zero16f

        def _filter_blk(i, cnt):
            idx = i * L + lane
            dvec = plsc.load_gather(ebuf_dst, [idx])
            svec = plsc.load_gather(ebuf_src, [idx])
            m = (dvec >= own_base) & (dvec < own_base + OWN)
            mi = m.astype(jnp.int32)
            pos = cnt + plsc.cumsum(mi) - 1
            plsc.store_scatter(sel_dst, [pos], dvec, mask=m)
            plsc.store_scatter(sel_src, [pos], svec, mask=m)
            return cnt + jnp.sum(mi)

        def _batch(b, carry):
            bb = b * L
            sv = plsc.load_gather(sel_src, [bb + lane])
            dv = plsc.load_gather(sel_dst, [bb + lane])
            lv = jnp.clip(dv - own_base, 0, OWN - 1)
            for k in range(8):
                idxbuf[pl.ds(k * L, L)] = sv * 8 + k
            cp_row = pltpu.async_copy(xw_hbm.at[idxbuf], rowbuf, sem_row)
            cp_a1 = pltpu.async_copy(aux_hbm.at[sv], aux1, sem_aux)
            cp_a2 = pltpu.async_copy(aux_hbm.at[dv], aux2, sem_aux)
            cp_a1.wait()
            cp_a2.wait()
            valid = (bb + lane) < carry
            ws = []
            for h in range(H):
                asv = plsc.load_gather(aux1, [lane, jnp.full((L,), h, jnp.int32)])
                adv = plsc.load_gather(aux2, [lane, jnp.full((L,), H + h, jnp.int32)])
                ssv = plsc.load_gather(aux2, [lane, jnp.full((L,), 2 * H + h, jnp.int32)])
                al = asv + adv
                al = jnp.maximum(al, 0.0) + 0.2 * jnp.minimum(al, 0.0)
                w = jnp.exp(al - ssv)
                ws.append(jnp.where(valid, w, 0.0))
            cp_row.wait()
            for e in range(L):
                re8 = jnp.sum(jnp.where(lane == e, lv, 0)) * 8
                wrow = zero16f
                for h in range(H):
                    wspl = jnp.take_along_axis(
                        ws[h], jnp.full((L,), e, jnp.int32), axis=0)
                    wrow = jnp.where(lane == h, wspl, wrow)
                    for q in range(C // L):
                        j = h * C + q * L
                        k, off = j // 128, j % 128
                        plsc.addupdate(accx.at[re8 + k, pl.ds(off, L)],
                                       rowbuf[k * L + e, pl.ds(off, L)] * wspl)
                re = lax.div(re8, 8)
                plsc.addupdate(accd.at[re, pl.ds(0, L)], wrow)
            return carry

        @pl.loop(0, NBLK)
        def _block(j):
            pltpu.sync_copy(src_hbm.at[pl.ds(j * EBLK, EBLK)], ebuf_src)
            pltpu.sync_copy(dst_hbm.at[pl.ds(j * EBLK, EBLK)], ebuf_dst)
            cnt = lax.fori_loop(0, EBLK // L, _filter_blk, jnp.int32(0),
                                unroll=False)
            nb = (cnt + (L - 1)) // L
            lax.fori_loop(0, nb, _batch, cnt, unroll=False)

        pltpu.sync_copy(accx, accx_out.at[pl.ds(own_base * 8, OWN * 8)])
        pltpu.sync_copy(accd, accd_out.at[pl.ds(own_base, OWN)])


def _sc_edge(xw, aux, src_e, dst_e):
    mesh = plsc.VectorSubcoreMesh(core_axis_name="c", subcore_axis_name="s")
    cp = pltpu.CompilerParams()
    if "needs_layout_passes" in pltpu.CompilerParams.__dataclass_fields__:
        cp = dataclasses.replace(cp, needs_layout_passes=False)
    f = pl.kernel(
        _sc_edge_body,
        compiler_params=cp,
        out_type=[
            jax.ShapeDtypeStruct((ROWS_PAD * 8, 128), jnp.float32),
            jax.ShapeDtypeStruct((ROWS_PAD, 128), jnp.float32),
        ],
        mesh=mesh,
        scratch_types=[
            pltpu.VMEM((EBLK,), jnp.int32),
            pltpu.VMEM((EBLK,), jnp.int32),
            pltpu.VMEM((SELCAP,), jnp.int32),
            pltpu.VMEM((SELCAP,), jnp.int32),
            pltpu.VMEM((8 * L, 128), jnp.float32),
            pltpu.VMEM((8 * L,), jnp.int32),
            pltpu.VMEM((L, AUXW), jnp.float32),
            pltpu.VMEM((L, AUXW), jnp.float32),
            pltpu.VMEM((L * H,), jnp.float32),
            pltpu.SemaphoreType.DMA,
            pltpu.SemaphoreType.DMA,
            pltpu.VMEM((OWN * 8, 128), jnp.float32),
            pltpu.VMEM((OWN, 128), jnp.float32),
        ],
    )
    return f(xw, aux, src_e, dst_e)


def _norm_body(accx_ref, accd_ref, bias_ref, pa_ref, out_ref):
    a = accx_ref[...]
    d = accd_ref[...]
    segs = []
    for h in range(H):
        r = 1.0 / d[:, h:h + 1]
        segs.append(a[:, h * C:(h + 1) * C] * r)
    o = jnp.concatenate(segs, axis=1) + bias_ref[...]
    pa = pa_ref[0, 0]
    out_ref[...] = jnp.where(o >= 0.0, o, pa * o)


def _norm_stage(accx, accd, bias_flat, pa):
    return pl.pallas_call(
        _norm_body,
        grid=(N // _ROW_BLK,),
        in_specs=[
            pl.BlockSpec((_ROW_BLK, HC), lambda i: (i, 0)),
            pl.BlockSpec((_ROW_BLK, 128), lambda i: (i, 0)),
            pl.BlockSpec((1, HC), lambda i: (0, 0)),
            pl.BlockSpec((1, 1), lambda i: (0, 0)),
        ],
        out_specs=pl.BlockSpec((_ROW_BLK, HC), lambda i: (i, 0)),
        out_shape=jax.ShapeDtypeStruct((N, HC), jnp.float32),
    )(accx, accd, bias_flat, pa)


def kernel(x, edge_index, W, att_src, att_dst, bias, prelu_a):
    xw, aux = _dense_stage(x, W, att_src.reshape(1, HC), att_dst.reshape(1, HC))
    xw2 = xw.reshape(ROWS_PAD * 8, 128)
    accx, accd = _sc_edge(xw2, aux, edge_index[0], edge_index[1])
    return _norm_stage(accx.reshape(ROWS_PAD, HC), accd, bias.reshape(1, HC),
                       jnp.reshape(prelu_a, (1, 1)))
